# baseline (speedup 1.0000x reference)
import functools

import jax
import jax.numpy as jnp
from jax import lax
from jax.experimental import pallas as pl
from jax.experimental.pallas import tpu as pltpu
from jax.experimental.pallas import tpu_sc as plsc

B, T, D, H, E = 2, 2048, 1024, 4096, 8
N = B * T
A = 2 * N
NW = 32
BM = 512
LOG_BM = 9
MAXT = A // BM + E
PAD = MAXT * BM
CH_A = A // NW
CH_S = PAD // NW
CH_T = N // NW

_SQRT_HALF = 0.7071067811865476


def _erf(z):
    a1, a2, a3, a4, a5 = (0.254829592, -0.284496736, 1.421413741,
                          -1.453152027, 1.061405429)
    p = 0.3275911
    az = jnp.abs(z)
    t = 1.0 / (1.0 + p * az)
    poly = ((((a5 * t + a4) * t + a3) * t + a2) * t + a1) * t
    e = 1.0 - poly * jnp.exp(-az * az)
    return jnp.sign(z) * e


def _gelu(x):
    return 0.5 * x * (1.0 + _erf(x * _SQRT_HALF))


def _gate_body(x_ref, gwt_ref, gb_ref, i_ref, v_ref, h_ref):
    logits = jnp.dot(x_ref[...], gwt_ref[...],
                     preferred_element_type=jnp.float32) + gb_ref[...]
    m = jnp.max(logits, axis=1, keepdims=True)
    el = jnp.exp(logits - m)
    p = el / jnp.sum(el, axis=1, keepdims=True)
    bn = p.shape[0]
    ii = lax.broadcasted_iota(jnp.int32, (bn, E), 1)
    m1 = jnp.max(p, axis=1, keepdims=True)
    i1 = jnp.min(jnp.where(p == m1, ii, E), axis=1, keepdims=True)
    oh1 = ii == i1
    p2 = jnp.where(oh1, -1.0, p)
    m2 = jnp.max(p2, axis=1, keepdims=True)
    i2 = jnp.min(jnp.where(p2 == m2, ii, E), axis=1, keepdims=True)
    oh2 = ii == i2
    i_ref[...] = jnp.concatenate([i1, i2], axis=1)
    v_ref[...] = jnp.concatenate([m1, m2], axis=1)
    ohsum = oh1.astype(jnp.int32) + oh2.astype(jnp.int32)
    subs = [jnp.sum(ohsum[k * 128:(k + 1) * 128], axis=0, keepdims=True)
            for k in range(4)]
    h_ref[0] = jnp.concatenate(subs, axis=0)


def _sc_sort_body(idx_hbm, val_hbm, hist_hbm, pos_hbm, tok_hbm, vs_hbm,
                  sched_hbm, idxv, valv, histl, posb, tokb, schedb):
    w = lax.axis_index("s") * 2 + lax.axis_index("c")
    pltpu.sync_copy(idx_hbm.at[w], idxv)
    pltpu.sync_copy(val_hbm.at[w], valv)
    pltpu.sync_copy(hist_hbm, histl)
    lane = lax.iota(jnp.int32, 16)
    zero16 = jnp.zeros((16,), jnp.int32)
    one16 = zero16 + 1
    wvec = zero16 + w
    tot = zero16
    before = zero16
    for wp in range(NW):
        row = histl[wp]
        tot = tot + row
        before = before + jnp.where(zero16 + wp < wvec, row, zero16)
    ntile = (tot + (BM - 1)) >> LOG_BM
    incl = jnp.cumsum(ntile)
    base = ((incl - ntile) << LOG_BM) + before
    total = jnp.sum(jnp.where(lane == (E - 1), incl, zero16))

    @pl.when(w == 0)
    def _sched():
        laste = 0
        for e in range(E - 1):
            incl_e = jnp.sum(jnp.where(lane == e, incl, zero16))
            laste = laste + jnp.where(total - 1 >= incl_e, 1, 0)
        for t2 in range(2):
            tv = lane + t2 * 16
            acc = zero16
            for e in range(E - 1):
                incl_e = jnp.sum(jnp.where(lane == e, incl, zero16))
                acc = acc + jnp.where(tv >= incl_e, one16, zero16)
            schedb[...] = jnp.where(tv >= total, laste, acc)
            pltpu.sync_copy(schedb, sched_hbm.at[pl.ds(t2 * 16, 16)])
        schedb[...] = jnp.where(lane == 0, total, 0)
        pltpu.sync_copy(schedb, sched_hbm.at[pl.ds(32, 16)])

    for v in range(16):
        ev = idxv[v // 8, pl.ds((v % 8) * 16, 16)]
        pos_v = zero16
        for e in range(E):
            m = ev == e
            mi = jnp.where(m, one16, zero16)
            rank = jnp.cumsum(mi) - mi
            s_base = jnp.sum(jnp.where(lane == e, base, zero16))
            pos_v = jnp.where(m, s_base + rank, pos_v)
            base = base + jnp.where(lane == e, jnp.sum(mi), 0)
        av = lane + (w * CH_A + v * 16)
        posb[v // 8, pl.ds((v % 8) * 16, 16)] = pos_v
        tokb[v // 8, pl.ds((v % 8) * 16, 16)] = lax.shift_right_logical(av, 1)
    pltpu.sync_copy(posb, pos_hbm.at[w])
    for j in range(2):
        pltpu.sync_copy(tokb.at[j], tok_hbm.at[posb.at[j]])
        pltpu.sync_copy(valv.at[j], vs_hbm.at[posb.at[j]])


def _sc_gather_body(tok_hbm, x_hbm, xs_hbm, tokb, idx0, idx1, row0, row1,
                    gs0, gs1, ws0, ws1):
    w = lax.axis_index("s") * 2 + lax.axis_index("c")
    pltpu.sync_copy(tok_hbm.at[pl.ds(w * CH_S, CH_S)], tokb)
    for i in range(CH_S // 16):
        t16 = tokb[pl.ds(i * 16, 16)]
        tokb[pl.ds(i * 16, 16)] = jnp.minimum(jnp.maximum(t16, 0), N - 1)
    bufs = [(idx0, row0, gs0, ws0), (idx1, row1, gs1, ws1)]
    nch = CH_S // 32
    gops = [None] * nch
    wops = [None] * nch
    for c in range(nch):
        ib, rb, gs, ws = bufs[c % 2]
        if c >= 2:
            wops[c - 2].wait()
        for j2 in range(2):
            ib[pl.ds(j2 * 16, 16)] = tokb[pl.ds(c * 32 + j2 * 16, 16)]
        gops[c] = pltpu.async_copy(x_hbm.at[ib], rb, gs)
        if c >= 1:
            pb, prb, pgs, pws = bufs[(c - 1) % 2]
            gops[c - 1].wait()
            wops[c - 1] = pltpu.async_copy(
                prb, xs_hbm.at[pl.ds(w * CH_S + (c - 1) * 32, 32)], pws)
    gops[nch - 1].wait()
    lb, lrb, lgs, lws = bufs[(nch - 1) % 2]
    wops[nch - 1] = pltpu.async_copy(
        lrb, xs_hbm.at[pl.ds(w * CH_S + (nch - 1) * 32, 32)], lws)
    wops[nch - 2].wait()
    wops[nch - 1].wait()


def _ffn_body(sched_ref, xs_ref, vs_ref, w1_ref, b1_ref, w2_ref, b2_ref,
              out_ref):
    t = pl.program_id(0)
    h = pl.program_id(1)
    total = sched_ref[32]

    @pl.when(t < total)
    def _compute():
        hblk = _gelu(jnp.dot(xs_ref[...], w1_ref[0],
                             preferred_element_type=jnp.float32)
                     + b1_ref[0, 0])
        contrib = jnp.dot(hblk.astype(jnp.bfloat16), w2_ref[0],
                          preferred_element_type=jnp.float32)
        vcol = vs_ref[0]

        @pl.when(h == 0)
        def _init():
            out_ref[...] = vcol * (contrib + b2_ref[0])

        @pl.when(h > 0)
        def _acc():
            out_ref[...] += vcol * contrib


def _sc_combine_body(pos_hbm, ys_hbm, out_hbm, posb, idx0, idx1, row0, row1,
                     out0, out1, gs0, gs1, ws0, ws1):
    w = lax.axis_index("s") * 2 + lax.axis_index("c")
    pltpu.sync_copy(pos_hbm.at[w], posb)
    bufs = [(idx0, row0, out0, gs0, ws0), (idx1, row1, out1, gs1, ws1)]
    nch = CH_A // 32
    gops = [None] * nch
    wops = [None] * nch

    def _emit(c):
        _, rb, ob, _, ws = bufs[c % 2]
        gops[c].wait()
        for tt in range(16):
            def _qb(qi, carry, tt=tt, rb=rb, ob=ob):
                off = qi * 64
                for u in range(4):
                    o = off + u * 16
                    ob[tt, pl.ds(o, 16)] = (rb[2 * tt, pl.ds(o, 16)]
                                            + rb[2 * tt + 1, pl.ds(o, 16)])
                return carry
            lax.fori_loop(0, D // 64, _qb, 0)
        wops[c] = pltpu.async_copy(
            ob, out_hbm.at[pl.ds(w * CH_T + c * 16, 16)], ws)

    for c in range(nch):
        ib, rb, ob, gs, ws = bufs[c % 2]
        if c >= 2:
            wops[c - 2].wait()
        for j2 in range(2):
            ib[pl.ds(j2 * 16, 16)] = posb[c // 4,
                                          pl.ds((c % 4) * 32 + j2 * 16, 16)]
        gops[c] = pltpu.async_copy(ys_hbm.at[ib], rb, gs)
        if c >= 1:
            _emit(c - 1)
    _emit(nch - 1)
    wops[nch - 2].wait()
    wops[nch - 1].wait()


def kernel(x, gate_w, gate_b, w1, b1, w2, b2):
    x_flat = x.reshape(N, D)
    BN = 512
    NT = N // BN
    BH = 512
    NH = H // BH

    top2i, top2v, hist = pl.pallas_call(
        _gate_body,
        grid=(NT,),
        in_specs=[
            pl.BlockSpec((BN, D), lambda n: (n, 0)),
            pl.BlockSpec((D, E), lambda n: (0, 0)),
            pl.BlockSpec((1, E), lambda n: (0, 0)),
        ],
        out_specs=[
            pl.BlockSpec((BN, 2), lambda n: (n, 0)),
            pl.BlockSpec((BN, 2), lambda n: (n, 0)),
            pl.BlockSpec((1, 4, E), lambda n: (n, 0, 0)),
        ],
        out_shape=[
            jax.ShapeDtypeStruct((N, 2), jnp.int32),
            jax.ShapeDtypeStruct((N, 2), jnp.float32),
            jax.ShapeDtypeStruct((NT, 4, E), jnp.int32),
        ],
    )(x_flat, gate_w.T, gate_b.reshape(1, E))

    idx3 = top2i.reshape(NW, 2, CH_A // 2)
    val3 = top2v.reshape(NW, 2, CH_A // 2)
    histp = jnp.pad(hist.reshape(NW, E), ((0, 0), (0, 16 - E)))

    mesh = plsc.VectorSubcoreMesh(core_axis_name="c", subcore_axis_name="s")

    scp = pltpu.CompilerParams(needs_layout_passes=False)
    sort_call = pl.kernel(
        _sc_sort_body, mesh=mesh, compiler_params=scp,
        out_type=[
            jax.ShapeDtypeStruct((NW, 2, CH_A // 2), jnp.int32),
            jax.ShapeDtypeStruct((PAD,), jnp.int32),
            jax.ShapeDtypeStruct((PAD,), jnp.float32),
            jax.ShapeDtypeStruct((48,), jnp.int32),
        ],
        scratch_types=[
            pltpu.VMEM((2, CH_A // 2), jnp.int32),
            pltpu.VMEM((2, CH_A // 2), jnp.float32),
            pltpu.VMEM((NW, 16), jnp.int32),
            pltpu.VMEM((2, CH_A // 2), jnp.int32),
            pltpu.VMEM((2, CH_A // 2), jnp.int32),
            pltpu.VMEM((16,), jnp.int32),
        ],
    )
    pos3, tokslot, valslot, sched = sort_call(idx3, val3, histp)

    x_i32 = lax.bitcast_convert_type(
        x_flat.astype(jnp.bfloat16).reshape(N, D // 2, 2), jnp.int32)
    gather_call = pl.kernel(
        _sc_gather_body, mesh=mesh, compiler_params=scp,
        out_type=jax.ShapeDtypeStruct((PAD, D // 2), jnp.int32),
        scratch_types=[
            pltpu.VMEM((CH_S,), jnp.int32),
            pltpu.VMEM((32,), jnp.int32),
            pltpu.VMEM((32,), jnp.int32),
            pltpu.VMEM((32, D // 2), jnp.int32),
            pltpu.VMEM((32, D // 2), jnp.int32),
            pltpu.SemaphoreType.DMA,
            pltpu.SemaphoreType.DMA,
            pltpu.SemaphoreType.DMA,
            pltpu.SemaphoreType.DMA,
        ],
    )
    xs_i32 = gather_call(tokslot, x_i32)
    xs = lax.bitcast_convert_type(
        xs_i32, jnp.bfloat16).reshape(PAD, D)

    grid_spec = pltpu.PrefetchScalarGridSpec(
        num_scalar_prefetch=1,
        grid=(MAXT, NH),
        in_specs=[
            pl.BlockSpec((BM, D), lambda t, h, sd: (t, 0)),
            pl.BlockSpec((1, BM, 1), lambda t, h, sd: (t, 0, 0)),
            pl.BlockSpec((1, D, BH), lambda t, h, sd: (sd[t], 0, h)),
            pl.BlockSpec((1, 1, 1, BH), lambda t, h, sd: (sd[t], h, 0, 0)),
            pl.BlockSpec((1, BH, D), lambda t, h, sd: (sd[t], h, 0)),
            pl.BlockSpec((1, 1, D), lambda t, h, sd: (sd[t], 0, 0)),
        ],
        out_specs=pl.BlockSpec((BM, D), lambda t, h, sd: (t, 0)),
    )
    ys = pl.pallas_call(
        _ffn_body,
        grid_spec=grid_spec,
        out_shape=jax.ShapeDtypeStruct((PAD, D), jnp.float32),
    )(sched, xs, valslot.reshape(MAXT, BM, 1), w1.astype(jnp.bfloat16),
      b1.reshape(E, NH, 1, BH), w2.astype(jnp.bfloat16),
      b2.reshape(E, 1, D))

    combine_call = pl.kernel(
        _sc_combine_body, mesh=mesh, compiler_params=scp,
        out_type=jax.ShapeDtypeStruct((N, D), jnp.float32),
        scratch_types=[
            pltpu.VMEM((2, CH_A // 2), jnp.int32),
            pltpu.VMEM((32,), jnp.int32),
            pltpu.VMEM((32,), jnp.int32),
            pltpu.VMEM((32, D), jnp.float32),
            pltpu.VMEM((32, D), jnp.float32),
            pltpu.VMEM((16, D), jnp.float32),
            pltpu.VMEM((16, D), jnp.float32),
            pltpu.SemaphoreType.DMA,
            pltpu.SemaphoreType.DMA,
            pltpu.SemaphoreType.DMA,
            pltpu.SemaphoreType.DMA,
        ],
    )
    out = combine_call(pos3, ys)

    return out.reshape(B, T, D)


# speedup vs baseline: 1.4980x; 1.4980x over previous
import functools

import jax
import jax.numpy as jnp
from jax import lax
from jax.experimental import pallas as pl
from jax.experimental.pallas import tpu as pltpu
from jax.experimental.pallas import tpu_sc as plsc

B, T, D, H, E = 2, 2048, 1024, 4096, 8
N = B * T
A = 2 * N
NW = 32
BM = 512
LOG_BM = 9
MAXT = A // BM + E
PAD = MAXT * BM
CH_A = A // NW
CH_S = PAD // NW
CH_T = N // NW

_SQRT_HALF = 0.7071067811865476


def _erf(z):
    a1, a2, a3, a4, a5 = (0.254829592, -0.284496736, 1.421413741,
                          -1.453152027, 1.061405429)
    p = 0.3275911
    az = jnp.abs(z)
    t = 1.0 / (1.0 + p * az)
    poly = ((((a5 * t + a4) * t + a3) * t + a2) * t + a1) * t
    e = 1.0 - poly * jnp.exp(-az * az)
    return jnp.sign(z) * e


def _gelu(x):
    return 0.5 * x * (1.0 + _erf(x * _SQRT_HALF))


def _gate_body(x_ref, gwt_ref, gb_ref, i_ref, v_ref, h_ref):
    logits = jnp.dot(x_ref[...], gwt_ref[...],
                     preferred_element_type=jnp.float32) + gb_ref[...]
    m = jnp.max(logits, axis=1, keepdims=True)
    el = jnp.exp(logits - m)
    p = el / jnp.sum(el, axis=1, keepdims=True)
    bn = p.shape[0]
    ii = lax.broadcasted_iota(jnp.int32, (bn, E), 1)
    m1 = jnp.max(p, axis=1, keepdims=True)
    i1 = jnp.min(jnp.where(p == m1, ii, E), axis=1, keepdims=True)
    oh1 = ii == i1
    p2 = jnp.where(oh1, -1.0, p)
    m2 = jnp.max(p2, axis=1, keepdims=True)
    i2 = jnp.min(jnp.where(p2 == m2, ii, E), axis=1, keepdims=True)
    oh2 = ii == i2
    i_ref[...] = jnp.concatenate([i1, i2], axis=1)
    v_ref[...] = jnp.concatenate([m1, m2], axis=1)
    ohsum = oh1.astype(jnp.int32) + oh2.astype(jnp.int32)
    subs = [jnp.sum(ohsum[k * 128:(k + 1) * 128], axis=0, keepdims=True)
            for k in range(4)]
    h_ref[0] = jnp.concatenate(subs, axis=0)


def _sc_sort_body(idx_hbm, val_hbm, hist_hbm, pos_hbm, tok_hbm, vs_hbm,
                  sched_hbm, idxv, valv, histl, posb, tokb, schedb):
    w = lax.axis_index("s") * 2 + lax.axis_index("c")
    pltpu.sync_copy(idx_hbm.at[w], idxv)
    pltpu.sync_copy(val_hbm.at[w], valv)
    pltpu.sync_copy(hist_hbm, histl)
    lane = lax.iota(jnp.int32, 16)
    zero16 = jnp.zeros((16,), jnp.int32)
    one16 = zero16 + 1
    wvec = zero16 + w
    tot = zero16
    before = zero16
    for wp in range(NW):
        row = histl[wp]
        tot = tot + row
        before = before + jnp.where(zero16 + wp < wvec, row, zero16)
    ntile = (tot + (BM - 1)) >> LOG_BM
    incl = jnp.cumsum(ntile)
    base = ((incl - ntile) << LOG_BM) + before
    total = jnp.sum(jnp.where(lane == (E - 1), incl, zero16))

    @pl.when(w == 0)
    def _sched():
        laste = 0
        for e in range(E - 1):
            incl_e = jnp.sum(jnp.where(lane == e, incl, zero16))
            laste = laste + jnp.where(total - 1 >= incl_e, 1, 0)
        for t2 in range(2):
            tv = lane + t2 * 16
            acc = zero16
            for e in range(E - 1):
                incl_e = jnp.sum(jnp.where(lane == e, incl, zero16))
                acc = acc + jnp.where(tv >= incl_e, one16, zero16)
            schedb[...] = jnp.where(tv >= total, laste, acc)
            pltpu.sync_copy(schedb, sched_hbm.at[pl.ds(t2 * 16, 16)])
        schedb[...] = jnp.where(lane == 0, total, 0)
        pltpu.sync_copy(schedb, sched_hbm.at[pl.ds(32, 16)])

    for v in range(16):
        ev = idxv[v // 8, pl.ds((v % 8) * 16, 16)]
        pos_v = zero16
        for e in range(E):
            m = ev == e
            mi = jnp.where(m, one16, zero16)
            rank = jnp.cumsum(mi) - mi
            s_base = jnp.sum(jnp.where(lane == e, base, zero16))
            pos_v = jnp.where(m, s_base + rank, pos_v)
            base = base + jnp.where(lane == e, jnp.sum(mi), 0)
        av = lane + (w * CH_A + v * 16)
        posb[v // 8, pl.ds((v % 8) * 16, 16)] = pos_v
        tokb[v // 8, pl.ds((v % 8) * 16, 16)] = lax.shift_right_logical(av, 1)
    pltpu.sync_copy(posb, pos_hbm.at[w])
    for j in range(2):
        pltpu.sync_copy(tokb.at[j], tok_hbm.at[posb.at[j]])
        pltpu.sync_copy(valv.at[j], vs_hbm.at[posb.at[j]])


def _sc_gather_body(tok_hbm, x_hbm, xs_hbm, tokb, idx0, idx1, row0, row1,
                    gs0, gs1, ws0, ws1):
    w = lax.axis_index("s") * 2 + lax.axis_index("c")
    pltpu.sync_copy(tok_hbm.at[pl.ds(w * CH_S, CH_S)], tokb)
    for i in range(CH_S // 16):
        t16 = tokb[pl.ds(i * 16, 16)]
        tokb[pl.ds(i * 16, 16)] = jnp.minimum(jnp.maximum(t16, 0), N - 1)
    bufs = [(idx0, row0, gs0, ws0), (idx1, row1, gs1, ws1)]
    bm = 48
    nch = CH_S // bm
    gops = [None] * nch
    wops = [None] * nch
    for c in range(nch):
        ib, rb, gs, ws = bufs[c % 2]
        if c >= 2:
            wops[c - 2].wait()
        for j2 in range(bm // 16):
            ib[pl.ds(j2 * 16, 16)] = tokb[pl.ds(c * bm + j2 * 16, 16)]
        gops[c] = pltpu.async_copy(x_hbm.at[ib], rb, gs)
        if c >= 1:
            pb, prb, pgs, pws = bufs[(c - 1) % 2]
            gops[c - 1].wait()
            wops[c - 1] = pltpu.async_copy(
                prb, xs_hbm.at[pl.ds(w * CH_S + (c - 1) * bm, bm)], pws)
    gops[nch - 1].wait()
    lb, lrb, lgs, lws = bufs[(nch - 1) % 2]
    wops[nch - 1] = pltpu.async_copy(
        lrb, xs_hbm.at[pl.ds(w * CH_S + (nch - 1) * bm, bm)], lws)
    wops[nch - 2].wait()
    wops[nch - 1].wait()


_C0 = 0.7978845608028654
_C1 = 0.044715


def _gelu_tanh(x):
    inner = _C0 * (x + _C1 * (x * x * x))
    return 0.5 * x * (1.0 + jnp.tanh(inner))


def _ffn_body(sched_ref, xs_ref, vs_ref, w1_ref, b1_ref, w2_ref, b2_ref,
              out_ref, xbf_ref):
    t = pl.program_id(0)
    h = pl.program_id(1)
    nh = pl.num_programs(1)
    total = sched_ref[32]

    @pl.when(t < total)
    def _compute():
        @pl.when(h == 0)
        def _cast():
            xbf_ref[...] = xs_ref[...].astype(jnp.bfloat16)

        d1 = jnp.dot(xbf_ref[...], w1_ref[0],
                     preferred_element_type=jnp.float32) + b1_ref[0, 0]
        hb = _gelu_tanh(d1.astype(jnp.bfloat16))
        contrib = jnp.dot(hb, w2_ref[0], preferred_element_type=jnp.float32)

        @pl.when(h == 0)
        def _init():
            out_ref[...] = contrib + b2_ref[0]

        @pl.when((h > 0) & (h < nh - 1))
        def _acc():
            out_ref[...] += contrib

        @pl.when(h == nh - 1)
        def _fin():
            out_ref[...] = vs_ref[0] * (out_ref[...] + contrib)


def _sc_combine_body(pos_hbm, ys_hbm, out_hbm, posb, idx0, idx1, row0, row1,
                     out0, out1, gs0, gs1, ws0, ws1):
    w = lax.axis_index("s") * 2 + lax.axis_index("c")
    pltpu.sync_copy(pos_hbm.at[w], posb)
    bufs = [(idx0, row0, out0, gs0, ws0), (idx1, row1, out1, gs1, ws1)]
    nch = CH_A // 32
    gops = [None] * nch
    wops = [None] * nch

    def _emit(c):
        _, rb, ob, _, ws = bufs[c % 2]
        gops[c].wait()
        for tt in range(16):
            def _qb(qi, carry, tt=tt, rb=rb, ob=ob):
                off = qi * 64
                for u in range(4):
                    o = off + u * 16
                    ob[tt, pl.ds(o, 16)] = (rb[2 * tt, pl.ds(o, 16)]
                                            + rb[2 * tt + 1, pl.ds(o, 16)])
                return carry
            lax.fori_loop(0, D // 64, _qb, 0)
        wops[c] = pltpu.async_copy(
            ob, out_hbm.at[pl.ds(w * CH_T + c * 16, 16)], ws)

    for c in range(nch):
        ib, rb, ob, gs, ws = bufs[c % 2]
        if c >= 2:
            wops[c - 2].wait()
        for j2 in range(2):
            ib[pl.ds(j2 * 16, 16)] = posb[c // 4,
                                          pl.ds((c % 4) * 32 + j2 * 16, 16)]
        gops[c] = pltpu.async_copy(ys_hbm.at[ib], rb, gs)
        if c >= 1:
            _emit(c - 1)
    _emit(nch - 1)
    wops[nch - 2].wait()
    wops[nch - 1].wait()


def kernel(x, gate_w, gate_b, w1, b1, w2, b2):
    x_flat = x.reshape(N, D)
    BN = 512
    NT = N // BN
    BH = 512
    NH = H // BH

    top2i, top2v, hist = pl.pallas_call(
        _gate_body,
        grid=(NT,),
        in_specs=[
            pl.BlockSpec((BN, D), lambda n: (n, 0)),
            pl.BlockSpec((D, E), lambda n: (0, 0)),
            pl.BlockSpec((1, E), lambda n: (0, 0)),
        ],
        out_specs=[
            pl.BlockSpec((BN, 2), lambda n: (n, 0)),
            pl.BlockSpec((BN, 2), lambda n: (n, 0)),
            pl.BlockSpec((1, 4, E), lambda n: (n, 0, 0)),
        ],
        out_shape=[
            jax.ShapeDtypeStruct((N, 2), jnp.int32),
            jax.ShapeDtypeStruct((N, 2), jnp.float32),
            jax.ShapeDtypeStruct((NT, 4, E), jnp.int32),
        ],
    )(x_flat, gate_w.T, gate_b.reshape(1, E))

    idx3 = top2i.reshape(NW, 2, CH_A // 2)
    val3 = top2v.reshape(NW, 2, CH_A // 2)
    histp = jnp.pad(hist.reshape(NW, E), ((0, 0), (0, 16 - E)))

    mesh = plsc.VectorSubcoreMesh(core_axis_name="c", subcore_axis_name="s")

    scp = pltpu.CompilerParams(needs_layout_passes=False)
    sort_call = pl.kernel(
        _sc_sort_body, mesh=mesh, compiler_params=scp,
        out_type=[
            jax.ShapeDtypeStruct((NW, 2, CH_A // 2), jnp.int32),
            jax.ShapeDtypeStruct((PAD,), jnp.int32),
            jax.ShapeDtypeStruct((PAD,), jnp.float32),
            jax.ShapeDtypeStruct((48,), jnp.int32),
        ],
        scratch_types=[
            pltpu.VMEM((2, CH_A // 2), jnp.int32),
            pltpu.VMEM((2, CH_A // 2), jnp.float32),
            pltpu.VMEM((NW, 16), jnp.int32),
            pltpu.VMEM((2, CH_A // 2), jnp.int32),
            pltpu.VMEM((2, CH_A // 2), jnp.int32),
            pltpu.VMEM((16,), jnp.int32),
        ],
    )
    pos3, tokslot, valslot, sched = sort_call(idx3, val3, histp)

    gather_call = pl.kernel(
        _sc_gather_body, mesh=mesh, compiler_params=scp,
        out_type=jax.ShapeDtypeStruct((PAD, D), jnp.float32),
        scratch_types=[
            pltpu.VMEM((CH_S,), jnp.int32),
            pltpu.VMEM((48,), jnp.int32),
            pltpu.VMEM((48,), jnp.int32),
            pltpu.VMEM((48, D), jnp.float32),
            pltpu.VMEM((48, D), jnp.float32),
            pltpu.SemaphoreType.DMA,
            pltpu.SemaphoreType.DMA,
            pltpu.SemaphoreType.DMA,
            pltpu.SemaphoreType.DMA,
        ],
    )
    xs = gather_call(tokslot, x_flat)

    grid_spec = pltpu.PrefetchScalarGridSpec(
        num_scalar_prefetch=1,
        grid=(MAXT, NH),
        in_specs=[
            pl.BlockSpec((BM, D), lambda t, h, sd: (t, 0)),
            pl.BlockSpec((1, BM, 1), lambda t, h, sd: (t, 0, 0)),
            pl.BlockSpec((1, D, BH), lambda t, h, sd: (sd[t], 0, h)),
            pl.BlockSpec((1, 1, 1, BH), lambda t, h, sd: (sd[t], h, 0, 0)),
            pl.BlockSpec((1, BH, D), lambda t, h, sd: (sd[t], h, 0)),
            pl.BlockSpec((1, 1, D), lambda t, h, sd: (sd[t], 0, 0)),
        ],
        out_specs=pl.BlockSpec((BM, D), lambda t, h, sd: (t, 0)),
        scratch_shapes=[pltpu.VMEM((BM, D), jnp.bfloat16)],
    )
    ys = pl.pallas_call(
        _ffn_body,
        grid_spec=grid_spec,
        out_shape=jax.ShapeDtypeStruct((PAD, D), jnp.float32),
    )(sched, xs, valslot.reshape(MAXT, BM, 1), w1.astype(jnp.bfloat16),
      b1.reshape(E, NH, 1, BH), w2.astype(jnp.bfloat16),
      b2.reshape(E, 1, D))

    combine_call = pl.kernel(
        _sc_combine_body, mesh=mesh, compiler_params=scp,
        out_type=jax.ShapeDtypeStruct((N, D), jnp.float32),
        scratch_types=[
            pltpu.VMEM((2, CH_A // 2), jnp.int32),
            pltpu.VMEM((32,), jnp.int32),
            pltpu.VMEM((32,), jnp.int32),
            pltpu.VMEM((32, D), jnp.float32),
            pltpu.VMEM((32, D), jnp.float32),
            pltpu.VMEM((16, D), jnp.float32),
            pltpu.VMEM((16, D), jnp.float32),
            pltpu.SemaphoreType.DMA,
            pltpu.SemaphoreType.DMA,
            pltpu.SemaphoreType.DMA,
            pltpu.SemaphoreType.DMA,
        ],
    )
    out = combine_call(pos3, ys)

    return out.reshape(B, T, D)
